# baseline (speedup 1.0000x reference)
import functools

import jax
import jax.numpy as jnp
from jax import lax
from jax.experimental import pallas as pl
from jax.experimental.pallas import tpu as pltpu
from jax.experimental.pallas import tpu_sc as plsc

N = 10000
E = 320000
F = 128
NC = 2
NS = 16
NW = NC * NS
EPW = E // NW
CH = 100
NCH = EPW // CH
NP = 10112
RPT = NP // NS
DEGW = 16
NPH = 4
NBUF = 2
ACH = 125
ANCH = EPW // ACH
ANCHB = ANCH // NPH

def _worker(c, s):
  return c * NS + s


@functools.cache
def _sc_kernels():
  mesh = plsc.VectorSubcoreMesh(core_axis_name="c", subcore_axis_name="s",
                                num_cores=NC, num_subcores=NS)

  @functools.partial(
      pl.kernel,
      out_type=jax.ShapeDtypeStruct((NC, NP, F), jnp.float32),
      mesh=mesh,
      scratch_types=[
          pltpu.VMEM((NCH, CH), jnp.int32),
          pltpu.VMEM((CH, F), jnp.float32),
          pltpu.VMEM_SHARED((NP, F), jnp.float32),
      ],
  )
  def deg_call(dst_hbm, zeros_hbm, ones_hbm, out_hbm, idx_v, ones_v, deg_sh):
    c = lax.axis_index("c")
    s = lax.axis_index("s")
    w = _worker(c, s)
    pltpu.sync_copy(ones_hbm, ones_v)
    pltpu.sync_copy(zeros_hbm, deg_sh.at[pl.ds(s * RPT, RPT)])
    pltpu.sync_copy(dst_hbm.at[w], idx_v)
    plsc.subcore_barrier()

    @pl.loop(0, NCH)
    def _chunk(j):
      pltpu.sync_copy(ones_v, deg_sh.at[idx_v.at[j]], add=True)

    plsc.subcore_barrier()
    pltpu.sync_copy(deg_sh.at[pl.ds(s * RPT, RPT)],
                    out_hbm.at[c, pl.ds(s * RPT, RPT)])

  @functools.partial(
      pl.kernel,
      out_type=jax.ShapeDtypeStruct((NC, NP, F), jnp.float32),
      mesh=mesh,
      scratch_types=[
          pltpu.VMEM((ANCHB, ACH), jnp.int32),
          pltpu.VMEM((ANCHB, ACH), jnp.int32),
          *([pltpu.VMEM((ACH, F), jnp.float32)] * NBUF),
          pltpu.VMEM_SHARED((NP, F), jnp.float32),
          *([pltpu.SemaphoreType.DMA] * NBUF),
      ],
  )
  def agg_call(m_hbm, src_hbm, dst_hbm, zeros_hbm, out_hbm,
               src_v, dst_v, *rest):
    rows = rest[:NBUF]
    agg_sh = rest[NBUF]
    sems = rest[NBUF + 1:]
    c = lax.axis_index("c")
    s = lax.axis_index("s")
    w = _worker(c, s)
    pltpu.sync_copy(zeros_hbm, agg_sh.at[pl.ds(s * RPT, RPT)])
    plsc.subcore_barrier()

    @pl.loop(0, NPH)
    def _phase(p):
      pltpu.sync_copy(src_hbm.at[w, p], src_v)
      pltpu.sync_copy(dst_hbm.at[w, p], dst_v)
      for b in range(NBUF):
        pltpu.async_copy(m_hbm.at[src_v.at[b]], rows[b], sems[b])

      @pl.loop(0, ANCHB, step=NBUF)
      def _chunk(j):
        for b in range(NBUF):
          k = j + b
          pltpu.make_async_copy(m_hbm.at[src_v.at[k]], rows[b], sems[b]).wait()
          pltpu.sync_copy(rows[b], agg_sh.at[dst_v.at[k]], add=True)
          kn = lax.rem(k + NBUF, ANCHB)
          pltpu.async_copy(m_hbm.at[src_v.at[kn]], rows[b], sems[b])

      for b in range(NBUF):
        pltpu.make_async_copy(m_hbm.at[src_v.at[b]], rows[b], sems[b]).wait()

    plsc.subcore_barrier()
    pltpu.sync_copy(agg_sh.at[pl.ds(s * RPT, RPT)],
                    out_hbm.at[c, pl.ds(s * RPT, RPT)])

  return deg_call, agg_call


BN = 400

_row_spec = pl.BlockSpec((BN, F), lambda i: (i, 0))
_deg_spec = pl.BlockSpec((BN, DEGW), lambda i: (i, 0))


def _full(shape):
  return pl.BlockSpec(shape, lambda i: tuple(0 for _ in shape))


def _hk_body(x_ref, w1t_ref, b1_ref, w2t_ref, b2_ref, h_ref):
  x = x_ref[...]
  h1 = jnp.maximum(jnp.dot(x, w1t_ref[...],
                           preferred_element_type=jnp.float32) + b1_ref[...], 0.0)
  h_ref[...] = jnp.maximum(
      jnp.dot(h1, w2t_ref[...],
              preferred_element_type=jnp.float32) + b2_ref[...], 0.0)


_hk = pl.pallas_call(
    _hk_body,
    grid=(N // BN,),
    in_specs=[_row_spec, _full((F, F)), _full((1, F)), _full((F, F)),
              _full((1, F))],
    out_specs=_row_spec,
    out_shape=jax.ShapeDtypeStruct((N, F), jnp.float32),
)


def _scale_body(h_ref, d0_ref, d1_ref, m0_ref, dinv_ref):
  deg = d0_ref[...][:, :1] + d1_ref[...][:, :1]
  dinv = lax.rsqrt(jnp.maximum(deg, 1.0))
  dinv_ref[...] = jnp.broadcast_to(dinv, (BN, DEGW))
  m0_ref[...] = h_ref[...] * dinv


_scale = pl.pallas_call(
    _scale_body,
    grid=(N // BN,),
    in_specs=[_row_spec, _row_spec, _row_spec],
    out_specs=[_row_spec, _deg_spec],
    out_shape=[jax.ShapeDtypeStruct((N, F), jnp.float32),
               jax.ShapeDtypeStruct((N, DEGW), jnp.float32)],
)


def _lap_body(h_ref, pa_ref, pb_ref, dinv_ref, lh_ref, m1_ref):
  dinv = dinv_ref[...][:, :1]
  lh = h_ref[...] - (pa_ref[...] + pb_ref[...]) * dinv
  lh_ref[...] = lh
  m1_ref[...] = lh * dinv


_lap = pl.pallas_call(
    _lap_body,
    grid=(N // BN,),
    in_specs=[_row_spec, _row_spec, _row_spec, _deg_spec],
    out_specs=[_row_spec, _row_spec],
    out_shape=[jax.ShapeDtypeStruct((N, F), jnp.float32),
               jax.ShapeDtypeStruct((N, F), jnp.float32)],
)


def _final_body(h_ref, lh_ref, pa_ref, pb_ref, dinv_ref, w3t_ref, b3_ref,
                w4t_ref, b4_ref, out_ref):
  dinv = dinv_ref[...][:, :1]
  lh = lh_ref[...]
  l2 = lh - (pa_ref[...] + pb_ref[...]) * dinv
  w3t = w3t_ref[...]
  b0 = w3t[0:F, :]
  b1 = w3t[F:2 * F, :]
  b2 = w3t[2 * F:3 * F, :]
  c0 = 3.0 * b0
  c1 = 3.0 * b1 - 3.0 * b0
  c2 = 0.75 * b0 - 1.5 * b1 + 0.75 * b2
  t = (jnp.dot(h_ref[...], c0, preferred_element_type=jnp.float32)
       + jnp.dot(lh, c1, preferred_element_type=jnp.float32)
       + jnp.dot(l2, c2, preferred_element_type=jnp.float32)
       + b3_ref[...])
  t = jnp.maximum(t, 0.0)
  out_ref[...] = jnp.dot(t, w4t_ref[...],
                         preferred_element_type=jnp.float32) + b4_ref[...]


_final = pl.pallas_call(
    _final_body,
    grid=(N // BN,),
    in_specs=[_row_spec, _row_spec, _row_spec, _row_spec, _deg_spec,
              _full((3 * F, F)), _full((1, F)), _full((F, 2)), _full((1, 2))],
    out_specs=pl.BlockSpec((BN, 2), lambda i: (i, 0)),
    out_shape=jax.ShapeDtypeStruct((N, 2), jnp.float32),
)


def kernel(in_feat, edge_index, W1, b1, W2, b2, W3, b3, W4, b4):
  src4 = edge_index[0].reshape(NW, NPH, ANCHB, ACH)
  dst4 = edge_index[1].reshape(NW, NPH, ANCHB, ACH)
  dst3 = edge_index[1].reshape(NW, NCH, CH)
  zeros_rows = jnp.zeros((RPT, F), jnp.float32)
  ones_rows = jnp.ones((CH, F), jnp.float32)

  deg_call, agg_call = _sc_kernels()
  degs = deg_call(dst3, zeros_rows, ones_rows)
  h = _hk(in_feat, W1.T, b1[None, :], W2.T, b2[None, :])
  m0, dinv = _scale(h, degs[0], degs[1])
  p0 = agg_call(m0, src4, dst4, zeros_rows)
  lh, m1 = _lap(h, p0[0], p0[1], dinv)
  p1 = agg_call(m1, src4, dst4, zeros_rows)
  return _final(h, lh, p1[0], p1[1], dinv, W3.T, b3[None, :], W4.T, b4[None, :])


# speedup vs baseline: 1.1237x; 1.1237x over previous
import functools

import jax
import jax.numpy as jnp
from jax import lax
from jax.experimental import pallas as pl
from jax.experimental.pallas import tpu as pltpu
from jax.experimental.pallas import tpu_sc as plsc

N = 10000
E = 320000
F = 128
NC = 2
NS = 16
NW = NC * NS
EPW = E // NW
CH = 100
NCH = EPW // CH
NP = 10112
RPT = NP // NS
DEGW = 16
NPH = 5
NBUF = 4
ACH = 50
ANCH = EPW // ACH
ANCHB = ANCH // NPH

def _worker(c, s):
  return c * NS + s


@functools.cache
def _sc_kernels():
  mesh = plsc.VectorSubcoreMesh(core_axis_name="c", subcore_axis_name="s",
                                num_cores=NC, num_subcores=NS)

  @functools.partial(
      pl.kernel,
      out_type=jax.ShapeDtypeStruct((NC, NP, F), jnp.float32),
      mesh=mesh,
      scratch_types=[
          pltpu.VMEM((NCH, CH), jnp.int32),
          pltpu.VMEM((CH, F), jnp.float32),
          pltpu.VMEM_SHARED((NP, F), jnp.float32),
      ],
  )
  def deg_call(dst_hbm, zeros_hbm, ones_hbm, out_hbm, idx_v, ones_v, deg_sh):
    c = lax.axis_index("c")
    s = lax.axis_index("s")
    w = _worker(c, s)
    pltpu.sync_copy(ones_hbm, ones_v)
    pltpu.sync_copy(zeros_hbm, deg_sh.at[pl.ds(s * RPT, RPT)])
    pltpu.sync_copy(dst_hbm.at[w], idx_v)
    plsc.subcore_barrier()

    @pl.loop(0, NCH)
    def _chunk(j):
      pltpu.sync_copy(ones_v, deg_sh.at[idx_v.at[j]], add=True)

    plsc.subcore_barrier()
    pltpu.sync_copy(deg_sh.at[pl.ds(s * RPT, RPT)],
                    out_hbm.at[c, pl.ds(s * RPT, RPT)])

  @functools.partial(
      pl.kernel,
      out_type=jax.ShapeDtypeStruct((NC, NP, F), jnp.float32),
      mesh=mesh,
      scratch_types=[
          pltpu.VMEM((ANCHB, ACH), jnp.int32),
          pltpu.VMEM((ANCHB, ACH), jnp.int32),
          *([pltpu.VMEM((ACH, F), jnp.float32)] * NBUF),
          pltpu.VMEM_SHARED((NP, F), jnp.float32),
          *([pltpu.SemaphoreType.DMA] * NBUF),
      ],
  )
  def agg_call(m_hbm, src_hbm, dst_hbm, zeros_hbm, out_hbm,
               src_v, dst_v, *rest):
    rows = rest[:NBUF]
    agg_sh = rest[NBUF]
    sems = rest[NBUF + 1:]
    c = lax.axis_index("c")
    s = lax.axis_index("s")
    w = _worker(c, s)
    pltpu.sync_copy(zeros_hbm, agg_sh.at[pl.ds(s * RPT, RPT)])
    plsc.subcore_barrier()

    @pl.loop(0, NPH)
    def _phase(p):
      pltpu.sync_copy(src_hbm.at[w, p], src_v)
      pltpu.sync_copy(dst_hbm.at[w, p], dst_v)
      for b in range(NBUF):
        pltpu.async_copy(m_hbm.at[src_v.at[b]], rows[b], sems[b])

      @pl.loop(0, ANCHB, step=NBUF)
      def _chunk(j):
        for b in range(NBUF):
          k = j + b
          pltpu.make_async_copy(m_hbm.at[src_v.at[k]], rows[b], sems[b]).wait()
          pltpu.sync_copy(rows[b], agg_sh.at[dst_v.at[k]], add=True)
          kn = lax.rem(k + NBUF, ANCHB)
          pltpu.async_copy(m_hbm.at[src_v.at[kn]], rows[b], sems[b])

      for b in range(NBUF):
        pltpu.make_async_copy(m_hbm.at[src_v.at[b]], rows[b], sems[b]).wait()

    plsc.subcore_barrier()
    pltpu.sync_copy(agg_sh.at[pl.ds(s * RPT, RPT)],
                    out_hbm.at[c, pl.ds(s * RPT, RPT)])

  return deg_call, agg_call


BN = 2000

_row_spec = pl.BlockSpec((BN, F), lambda i: (i, 0))
_deg_spec = pl.BlockSpec((BN, DEGW), lambda i: (i, 0))


def _full(shape):
  return pl.BlockSpec(shape, lambda i: tuple(0 for _ in shape))


def _hk_body(x_ref, w1t_ref, b1_ref, w2t_ref, b2_ref, h_ref):
  x = x_ref[...]
  h1 = jnp.maximum(jnp.dot(x, w1t_ref[...],
                           preferred_element_type=jnp.float32) + b1_ref[...], 0.0)
  h_ref[...] = jnp.maximum(
      jnp.dot(h1, w2t_ref[...],
              preferred_element_type=jnp.float32) + b2_ref[...], 0.0)


_hk = pl.pallas_call(
    _hk_body,
    grid=(N // BN,),
    in_specs=[_row_spec, _full((F, F)), _full((1, F)), _full((F, F)),
              _full((1, F))],
    out_specs=_row_spec,
    out_shape=jax.ShapeDtypeStruct((N, F), jnp.float32),
)


def _scale_body(h_ref, d0_ref, d1_ref, m0_ref, dinv_ref):
  deg = d0_ref[...][:, :1] + d1_ref[...][:, :1]
  dinv = lax.rsqrt(jnp.maximum(deg, 1.0))
  dinv_ref[...] = jnp.broadcast_to(dinv, (BN, DEGW))
  m0_ref[...] = h_ref[...] * dinv


_scale = pl.pallas_call(
    _scale_body,
    grid=(N // BN,),
    in_specs=[_row_spec, _row_spec, _row_spec],
    out_specs=[_row_spec, _deg_spec],
    out_shape=[jax.ShapeDtypeStruct((N, F), jnp.float32),
               jax.ShapeDtypeStruct((N, DEGW), jnp.float32)],
)


def _lap_body(h_ref, pa_ref, pb_ref, dinv_ref, lh_ref, m1_ref):
  dinv = dinv_ref[...][:, :1]
  lh = h_ref[...] - (pa_ref[...] + pb_ref[...]) * dinv
  lh_ref[...] = lh
  m1_ref[...] = lh * dinv


_lap = pl.pallas_call(
    _lap_body,
    grid=(N // BN,),
    in_specs=[_row_spec, _row_spec, _row_spec, _deg_spec],
    out_specs=[_row_spec, _row_spec],
    out_shape=[jax.ShapeDtypeStruct((N, F), jnp.float32),
               jax.ShapeDtypeStruct((N, F), jnp.float32)],
)


def _final_body(h_ref, lh_ref, pa_ref, pb_ref, dinv_ref, w3t_ref, b3_ref,
                w4t_ref, b4_ref, out_ref):
  dinv = dinv_ref[...][:, :1]
  lh = lh_ref[...]
  l2 = lh - (pa_ref[...] + pb_ref[...]) * dinv
  w3t = w3t_ref[...]
  b0 = w3t[0:F, :]
  b1 = w3t[F:2 * F, :]
  b2 = w3t[2 * F:3 * F, :]
  c0 = 3.0 * b0
  c1 = 3.0 * b1 - 3.0 * b0
  c2 = 0.75 * b0 - 1.5 * b1 + 0.75 * b2
  t = (jnp.dot(h_ref[...], c0, preferred_element_type=jnp.float32)
       + jnp.dot(lh, c1, preferred_element_type=jnp.float32)
       + jnp.dot(l2, c2, preferred_element_type=jnp.float32)
       + b3_ref[...])
  t = jnp.maximum(t, 0.0)
  out_ref[...] = jnp.dot(t, w4t_ref[...],
                         preferred_element_type=jnp.float32) + b4_ref[...]


_final = pl.pallas_call(
    _final_body,
    grid=(N // BN,),
    in_specs=[_row_spec, _row_spec, _row_spec, _row_spec, _deg_spec,
              _full((3 * F, F)), _full((1, F)), _full((F, 2)), _full((1, 2))],
    out_specs=pl.BlockSpec((BN, 2), lambda i: (i, 0)),
    out_shape=jax.ShapeDtypeStruct((N, 2), jnp.float32),
)


def kernel(in_feat, edge_index, W1, b1, W2, b2, W3, b3, W4, b4):
  src4 = edge_index[0].reshape(NW, NPH, ANCHB, ACH)
  dst4 = edge_index[1].reshape(NW, NPH, ANCHB, ACH)
  dst3 = edge_index[1].reshape(NW, NCH, CH)
  zeros_rows = jnp.zeros((RPT, F), jnp.float32)
  ones_rows = jnp.ones((CH, F), jnp.float32)

  deg_call, agg_call = _sc_kernels()
  degs = deg_call(dst3, zeros_rows, ones_rows)
  h = _hk(in_feat, W1.T, b1[None, :], W2.T, b2[None, :])
  m0, dinv = _scale(h, degs[0], degs[1])
  p0 = agg_call(m0, src4, dst4, zeros_rows)
  lh, m1 = _lap(h, p0[0], p0[1], dinv)
  p1 = agg_call(m1, src4, dst4, zeros_rows)
  return _final(h, lh, p1[0], p1[1], dinv, W3.T, b3[None, :], W4.T, b4[None, :])


# speedup vs baseline: 1.1298x; 1.0055x over previous
import functools

import jax
import jax.numpy as jnp
from jax import lax
from jax.experimental import pallas as pl
from jax.experimental.pallas import tpu as pltpu
from jax.experimental.pallas import tpu_sc as plsc

N = 10000
E = 320000
F = 128
NC = 2
NS = 16
NW = NC * NS
EPW = E // NW
CH = 100
NCH = EPW // CH
NP = 10112
RPT = NP // NS
DEGW = 16
NPH = 5
NBUF = 4
ACH = 50
ANCH = EPW // ACH
ANCHB = ANCH // NPH

def _worker(c, s):
  return c * NS + s


@functools.cache
def _sc_kernels():
  mesh = plsc.VectorSubcoreMesh(core_axis_name="c", subcore_axis_name="s",
                                num_cores=NC, num_subcores=NS)

  @functools.partial(
      pl.kernel,
      out_type=jax.ShapeDtypeStruct((NC, NP, F), jnp.float32),
      mesh=mesh,
      scratch_types=[
          pltpu.VMEM((NCH, CH), jnp.int32),
          pltpu.VMEM((CH, F), jnp.float32),
          pltpu.VMEM_SHARED((NP, F), jnp.float32),
      ],
  )
  def deg_call(dst_hbm, zeros_hbm, ones_hbm, out_hbm, idx_v, ones_v, deg_sh):
    c = lax.axis_index("c")
    s = lax.axis_index("s")
    w = _worker(c, s)
    pltpu.sync_copy(ones_hbm, ones_v)
    pltpu.sync_copy(zeros_hbm, deg_sh.at[pl.ds(s * RPT, RPT)])
    pltpu.sync_copy(dst_hbm.at[w], idx_v)
    plsc.subcore_barrier()

    @pl.loop(0, NCH)
    def _chunk(j):
      pltpu.sync_copy(ones_v, deg_sh.at[idx_v.at[j]], add=True)

    plsc.subcore_barrier()
    pltpu.sync_copy(deg_sh.at[pl.ds(s * RPT, RPT)],
                    out_hbm.at[c, pl.ds(s * RPT, RPT)])

  @functools.partial(
      pl.kernel,
      out_type=jax.ShapeDtypeStruct((NC, NP, F), jnp.float32),
      mesh=mesh,
      scratch_types=[
          pltpu.VMEM((ANCHB, ACH), jnp.int32),
          pltpu.VMEM((ANCHB, ACH), jnp.int32),
          *([pltpu.VMEM((ACH, F), jnp.float32)] * NBUF),
          pltpu.VMEM_SHARED((NP, F), jnp.float32),
          *([pltpu.SemaphoreType.DMA] * NBUF),
      ],
  )
  def agg_call(m_hbm, src_hbm, dst_hbm, zeros_hbm, out_hbm,
               src_v, dst_v, *rest):
    rows = rest[:NBUF]
    agg_sh = rest[NBUF]
    sems = rest[NBUF + 1:]
    c = lax.axis_index("c")
    s = lax.axis_index("s")
    w = _worker(c, s)
    pltpu.sync_copy(zeros_hbm, agg_sh.at[pl.ds(s * RPT, RPT)])
    plsc.subcore_barrier()

    @pl.loop(0, NPH)
    def _phase(p):
      pltpu.sync_copy(src_hbm.at[w, p], src_v)
      pltpu.sync_copy(dst_hbm.at[w, p], dst_v)
      for b in range(NBUF):
        pltpu.async_copy(m_hbm.at[src_v.at[b]], rows[b], sems[b])

      @pl.loop(0, ANCHB, step=NBUF)
      def _chunk(j):
        for b in range(NBUF):
          k = j + b
          pltpu.make_async_copy(m_hbm.at[src_v.at[k]], rows[b], sems[b]).wait()
          pltpu.sync_copy(rows[b], agg_sh.at[dst_v.at[k]], add=True)
          kn = lax.rem(k + NBUF, ANCHB)
          pltpu.async_copy(m_hbm.at[src_v.at[kn]], rows[b], sems[b])

      for b in range(NBUF):
        pltpu.make_async_copy(m_hbm.at[src_v.at[b]], rows[b], sems[b]).wait()

    plsc.subcore_barrier()
    pltpu.sync_copy(agg_sh.at[pl.ds(s * RPT, RPT)],
                    out_hbm.at[c, pl.ds(s * RPT, RPT)])

  return deg_call, agg_call


BN = 5000

_row_spec = pl.BlockSpec((BN, F), lambda i: (i, 0))
_deg_spec = pl.BlockSpec((BN, DEGW), lambda i: (i, 0))


def _full(shape):
  return pl.BlockSpec(shape, lambda i: tuple(0 for _ in shape))


def _hk_body(x_ref, w1t_ref, b1_ref, w2t_ref, b2_ref, h_ref):
  x = x_ref[...]
  h1 = jnp.maximum(jnp.dot(x, w1t_ref[...],
                           preferred_element_type=jnp.float32) + b1_ref[...], 0.0)
  h_ref[...] = jnp.maximum(
      jnp.dot(h1, w2t_ref[...],
              preferred_element_type=jnp.float32) + b2_ref[...], 0.0)


_hk = pl.pallas_call(
    _hk_body,
    grid=(N // BN,),
    in_specs=[_row_spec, _full((F, F)), _full((1, F)), _full((F, F)),
              _full((1, F))],
    out_specs=_row_spec,
    out_shape=jax.ShapeDtypeStruct((N, F), jnp.float32),
)


def _scale_body(h_ref, d0_ref, d1_ref, m0_ref, dinv_ref):
  deg = d0_ref[...][:, :1] + d1_ref[...][:, :1]
  dinv = lax.rsqrt(jnp.maximum(deg, 1.0))
  dinv_ref[...] = jnp.broadcast_to(dinv, (BN, DEGW))
  m0_ref[...] = h_ref[...] * dinv


_scale = pl.pallas_call(
    _scale_body,
    grid=(N // BN,),
    in_specs=[_row_spec, _row_spec, _row_spec],
    out_specs=[_row_spec, _deg_spec],
    out_shape=[jax.ShapeDtypeStruct((N, F), jnp.float32),
               jax.ShapeDtypeStruct((N, DEGW), jnp.float32)],
)


def _lap_body(h_ref, pa_ref, pb_ref, dinv_ref, lh_ref, m1_ref):
  dinv = dinv_ref[...][:, :1]
  lh = h_ref[...] - (pa_ref[...] + pb_ref[...]) * dinv
  lh_ref[...] = lh
  m1_ref[...] = lh * dinv


_lap = pl.pallas_call(
    _lap_body,
    grid=(N // BN,),
    in_specs=[_row_spec, _row_spec, _row_spec, _deg_spec],
    out_specs=[_row_spec, _row_spec],
    out_shape=[jax.ShapeDtypeStruct((N, F), jnp.float32),
               jax.ShapeDtypeStruct((N, F), jnp.float32)],
)


def _final_body(h_ref, lh_ref, pa_ref, pb_ref, dinv_ref, w3t_ref, b3_ref,
                w4t_ref, b4_ref, out_ref):
  dinv = dinv_ref[...][:, :1]
  lh = lh_ref[...]
  l2 = lh - (pa_ref[...] + pb_ref[...]) * dinv
  w3t = w3t_ref[...]
  b0 = w3t[0:F, :]
  b1 = w3t[F:2 * F, :]
  b2 = w3t[2 * F:3 * F, :]
  c0 = 3.0 * b0
  c1 = 3.0 * b1 - 3.0 * b0
  c2 = 0.75 * b0 - 1.5 * b1 + 0.75 * b2
  t = (jnp.dot(h_ref[...], c0, preferred_element_type=jnp.float32)
       + jnp.dot(lh, c1, preferred_element_type=jnp.float32)
       + jnp.dot(l2, c2, preferred_element_type=jnp.float32)
       + b3_ref[...])
  t = jnp.maximum(t, 0.0)
  out_ref[...] = jnp.dot(t, w4t_ref[...],
                         preferred_element_type=jnp.float32) + b4_ref[...]


_final = pl.pallas_call(
    _final_body,
    grid=(N // BN,),
    in_specs=[_row_spec, _row_spec, _row_spec, _row_spec, _deg_spec,
              _full((3 * F, F)), _full((1, F)), _full((F, 2)), _full((1, 2))],
    out_specs=pl.BlockSpec((BN, 2), lambda i: (i, 0)),
    out_shape=jax.ShapeDtypeStruct((N, 2), jnp.float32),
)


def kernel(in_feat, edge_index, W1, b1, W2, b2, W3, b3, W4, b4):
  src4 = edge_index[0].reshape(NW, NPH, ANCHB, ACH)
  dst4 = edge_index[1].reshape(NW, NPH, ANCHB, ACH)
  dst3 = edge_index[1].reshape(NW, NCH, CH)
  zeros_rows = jnp.zeros((RPT, F), jnp.float32)
  ones_rows = jnp.ones((CH, F), jnp.float32)

  deg_call, agg_call = _sc_kernels()
  degs = deg_call(dst3, zeros_rows, ones_rows)
  h = _hk(in_feat, W1.T, b1[None, :], W2.T, b2[None, :])
  m0, dinv = _scale(h, degs[0], degs[1])
  p0 = agg_call(m0, src4, dst4, zeros_rows)
  lh, m1 = _lap(h, p0[0], p0[1], dinv)
  p1 = agg_call(m1, src4, dst4, zeros_rows)
  return _final(h, lh, p1[0], p1[1], dinv, W3.T, b3[None, :], W4.T, b4[None, :])


# speedup vs baseline: 1.2033x; 1.0651x over previous
import functools

import jax
import jax.numpy as jnp
from jax import lax
from jax.experimental import pallas as pl
from jax.experimental.pallas import tpu as pltpu
from jax.experimental.pallas import tpu_sc as plsc

N = 10000
E = 320000
F = 128
NC = 2
NS = 16
NW = NC * NS
EPW = E // NW
CH = 100
NCH = EPW // CH
NP = 10112
RPT = NP // NS
DEGW = 16
NPH = 10
NBUF = 4
ACH = 50
ANCH = EPW // ACH
ANCHB = ANCH // NPH

def _worker(c, s):
  return c * NS + s


@functools.cache
def _sc_kernels():
  mesh = plsc.VectorSubcoreMesh(core_axis_name="c", subcore_axis_name="s",
                                num_cores=NC, num_subcores=NS)

  @functools.partial(
      pl.kernel,
      out_type=jax.ShapeDtypeStruct((NC, NP, F), jnp.float32),
      mesh=mesh,
      scratch_types=[
          pltpu.VMEM((NCH, CH), jnp.int32),
          pltpu.VMEM((CH, F), jnp.float32),
          pltpu.VMEM_SHARED((NP, F), jnp.float32),
      ],
  )
  def deg_call(dst_hbm, zeros_hbm, ones_hbm, out_hbm, idx_v, ones_v, deg_sh):
    c = lax.axis_index("c")
    s = lax.axis_index("s")
    w = _worker(c, s)
    pltpu.sync_copy(ones_hbm, ones_v)
    pltpu.sync_copy(zeros_hbm, deg_sh.at[pl.ds(s * RPT, RPT)])
    pltpu.sync_copy(dst_hbm.at[w], idx_v)
    plsc.subcore_barrier()

    @pl.loop(0, NCH)
    def _chunk(j):
      pltpu.sync_copy(ones_v, deg_sh.at[idx_v.at[j]], add=True)

    plsc.subcore_barrier()
    pltpu.sync_copy(deg_sh.at[pl.ds(s * RPT, RPT)],
                    out_hbm.at[c, pl.ds(s * RPT, RPT)])

  @functools.partial(
      pl.kernel,
      out_type=jax.ShapeDtypeStruct((NC, NP, F), jnp.float32),
      mesh=mesh,
      scratch_types=[
          pltpu.VMEM((ANCHB, ACH), jnp.int32),
          pltpu.VMEM((ANCHB, ACH), jnp.int32),
          pltpu.VMEM((ANCHB, ACH), jnp.int32),
          pltpu.VMEM((ANCHB, ACH), jnp.int32),
          *([pltpu.VMEM((ACH, F), jnp.float32)] * NBUF),
          pltpu.VMEM_SHARED((NP, F), jnp.float32),
          pltpu.SemaphoreType.DMA,
          pltpu.SemaphoreType.DMA,
          *([pltpu.SemaphoreType.DMA] * NBUF),
      ],
  )
  def agg_call(m_hbm, src_hbm, dst_hbm, zeros_hbm, out_hbm, *scr):
    src_g = (scr[0], scr[2])
    dst_g = (scr[1], scr[3])
    rows = scr[4:4 + NBUF]
    agg_sh = scr[4 + NBUF]
    isem = (scr[5 + NBUF], scr[6 + NBUF])
    sems = scr[7 + NBUF:]
    c = lax.axis_index("c")
    s = lax.axis_index("s")
    w = _worker(c, s)
    pltpu.sync_copy(zeros_hbm, agg_sh.at[pl.ds(s * RPT, RPT)])
    plsc.subcore_barrier()

    pltpu.sync_copy(src_hbm.at[w, 0], src_g[0])
    pltpu.sync_copy(dst_hbm.at[w, 0], dst_g[0])
    pltpu.async_copy(src_hbm.at[w, 1], src_g[1], isem[1])
    pltpu.async_copy(dst_hbm.at[w, 1], dst_g[1], isem[1])
    for b in range(NBUF):
      pltpu.async_copy(m_hbm.at[src_g[0].at[b]], rows[b], sems[b])

    @pl.loop(0, NPH, step=2)
    def _phase(p):
      for q in (0, 1):
        src_v, dst_v = src_g[q], dst_g[q]

        @pl.loop(0, ANCHB - NBUF, step=NBUF)
        def _chunk(j):
          for b in range(NBUF):
            k = j + b
            pltpu.make_async_copy(m_hbm.at[src_v.at[k]], rows[b],
                                  sems[b]).wait()
            pltpu.sync_copy(rows[b], agg_sh.at[dst_v.at[k]], add=True)
            pltpu.async_copy(m_hbm.at[src_v.at[k + NBUF]], rows[b], sems[b])

        nsrc = src_g[1 - q]
        pltpu.make_async_copy(src_hbm.at[w, 0], nsrc, isem[1 - q]).wait()
        pltpu.make_async_copy(dst_hbm.at[w, 0], dst_g[1 - q],
                              isem[1 - q]).wait()

        for b in range(NBUF):
          k = ANCHB - NBUF + b
          pltpu.make_async_copy(m_hbm.at[src_v.at[k]], rows[b], sems[b]).wait()
          pltpu.sync_copy(rows[b], agg_sh.at[dst_v.at[k]], add=True)
          pltpu.async_copy(m_hbm.at[nsrc.at[b]], rows[b], sems[b])

        pn = jnp.minimum(p + q + 2, NPH - 1)
        pltpu.async_copy(src_hbm.at[w, pn], src_v, isem[q])
        pltpu.async_copy(dst_hbm.at[w, pn], dst_v, isem[q])

    for b in range(NBUF):
      pltpu.make_async_copy(m_hbm.at[src_g[0].at[b]], rows[b], sems[b]).wait()
    pltpu.make_async_copy(src_hbm.at[w, 0], src_g[1], isem[1]).wait()
    pltpu.make_async_copy(dst_hbm.at[w, 0], dst_g[1], isem[1]).wait()

    plsc.subcore_barrier()
    pltpu.sync_copy(agg_sh.at[pl.ds(s * RPT, RPT)],
                    out_hbm.at[c, pl.ds(s * RPT, RPT)])

  return deg_call, agg_call


BN = 5000

_row_spec = pl.BlockSpec((BN, F), lambda i: (i, 0))
_deg_spec = pl.BlockSpec((BN, DEGW), lambda i: (i, 0))


def _full(shape):
  return pl.BlockSpec(shape, lambda i: tuple(0 for _ in shape))


def _hk_body(x_ref, w1t_ref, b1_ref, w2t_ref, b2_ref, h_ref):
  x = x_ref[...]
  h1 = jnp.maximum(jnp.dot(x, w1t_ref[...],
                           preferred_element_type=jnp.float32) + b1_ref[...], 0.0)
  h_ref[...] = jnp.maximum(
      jnp.dot(h1, w2t_ref[...],
              preferred_element_type=jnp.float32) + b2_ref[...], 0.0)


_hk = pl.pallas_call(
    _hk_body,
    grid=(N // BN,),
    in_specs=[_row_spec, _full((F, F)), _full((1, F)), _full((F, F)),
              _full((1, F))],
    out_specs=_row_spec,
    out_shape=jax.ShapeDtypeStruct((N, F), jnp.float32),
)


def _scale_body(h_ref, d0_ref, d1_ref, m0_ref, dinv_ref):
  deg = d0_ref[...][:, :1] + d1_ref[...][:, :1]
  dinv = lax.rsqrt(jnp.maximum(deg, 1.0))
  dinv_ref[...] = jnp.broadcast_to(dinv, (BN, DEGW))
  m0_ref[...] = h_ref[...] * dinv


_scale = pl.pallas_call(
    _scale_body,
    grid=(N // BN,),
    in_specs=[_row_spec, _row_spec, _row_spec],
    out_specs=[_row_spec, _deg_spec],
    out_shape=[jax.ShapeDtypeStruct((N, F), jnp.float32),
               jax.ShapeDtypeStruct((N, DEGW), jnp.float32)],
)


def _lap_body(h_ref, pa_ref, pb_ref, dinv_ref, lh_ref, m1_ref):
  dinv = dinv_ref[...][:, :1]
  lh = h_ref[...] - (pa_ref[...] + pb_ref[...]) * dinv
  lh_ref[...] = lh
  m1_ref[...] = lh * dinv


_lap = pl.pallas_call(
    _lap_body,
    grid=(N // BN,),
    in_specs=[_row_spec, _row_spec, _row_spec, _deg_spec],
    out_specs=[_row_spec, _row_spec],
    out_shape=[jax.ShapeDtypeStruct((N, F), jnp.float32),
               jax.ShapeDtypeStruct((N, F), jnp.float32)],
)


def _final_body(h_ref, lh_ref, pa_ref, pb_ref, dinv_ref, w3t_ref, b3_ref,
                w4t_ref, b4_ref, out_ref):
  dinv = dinv_ref[...][:, :1]
  lh = lh_ref[...]
  l2 = lh - (pa_ref[...] + pb_ref[...]) * dinv
  w3t = w3t_ref[...]
  b0 = w3t[0:F, :]
  b1 = w3t[F:2 * F, :]
  b2 = w3t[2 * F:3 * F, :]
  c0 = 3.0 * b0
  c1 = 3.0 * b1 - 3.0 * b0
  c2 = 0.75 * b0 - 1.5 * b1 + 0.75 * b2
  t = (jnp.dot(h_ref[...], c0, preferred_element_type=jnp.float32)
       + jnp.dot(lh, c1, preferred_element_type=jnp.float32)
       + jnp.dot(l2, c2, preferred_element_type=jnp.float32)
       + b3_ref[...])
  t = jnp.maximum(t, 0.0)
  out_ref[...] = jnp.dot(t, w4t_ref[...],
                         preferred_element_type=jnp.float32) + b4_ref[...]


_final = pl.pallas_call(
    _final_body,
    grid=(N // BN,),
    in_specs=[_row_spec, _row_spec, _row_spec, _row_spec, _deg_spec,
              _full((3 * F, F)), _full((1, F)), _full((F, 2)), _full((1, 2))],
    out_specs=pl.BlockSpec((BN, 2), lambda i: (i, 0)),
    out_shape=jax.ShapeDtypeStruct((N, 2), jnp.float32),
)


def kernel(in_feat, edge_index, W1, b1, W2, b2, W3, b3, W4, b4):
  src4 = edge_index[0].reshape(NW, NPH, ANCHB, ACH)
  dst4 = edge_index[1].reshape(NW, NPH, ANCHB, ACH)
  dst3 = edge_index[1].reshape(NW, NCH, CH)
  zeros_rows = jnp.zeros((RPT, F), jnp.float32)
  ones_rows = jnp.ones((CH, F), jnp.float32)

  deg_call, agg_call = _sc_kernels()
  degs = deg_call(dst3, zeros_rows, ones_rows)
  h = _hk(in_feat, W1.T, b1[None, :], W2.T, b2[None, :])
  m0, dinv = _scale(h, degs[0], degs[1])
  p0 = agg_call(m0, src4, dst4, zeros_rows)
  lh, m1 = _lap(h, p0[0], p0[1], dinv)
  p1 = agg_call(m1, src4, dst4, zeros_rows)
  return _final(h, lh, p1[0], p1[1], dinv, W3.T, b3[None, :], W4.T, b4[None, :])
